# speedup vs baseline: 1.2530x; 1.2530x over previous
import functools

import jax
import jax.numpy as jnp
from jax import lax
from jax.experimental import pallas as pl
from jax.experimental.pallas import tpu as pltpu
from jax.experimental.pallas import tpu_sc as plsc

D_MODEL = 1024
BATCH = 16384
LANES = 16
NUM_CORES = 2
NUM_SUBCORES = 16
NUM_WORKERS = NUM_CORES * NUM_SUBCORES
B_PER_W = BATCH // NUM_WORKERS
CHUNK = 32
NUM_CHUNKS = B_PER_W // CHUNK


def _sc_kernel(src_w, dst_w, mov1, mov2, out, idx1_v, idx2_v, buf1, buf2,
               sem1, sem2):
    wid = lax.axis_index("s") * NUM_CORES + lax.axis_index("c")
    base = wid * B_PER_W

    pltpu.sync_copy(mov1.at[pl.ds(base, B_PER_W)], idx1_v)
    pltpu.sync_copy(mov2.at[pl.ds(base, B_PER_W)], idx2_v)

    def chunk_body(k, carry):
        off = pl.multiple_of(k * CHUNK, CHUNK)
        cp1 = pltpu.async_copy(src_w.at[idx1_v.at[pl.ds(off, CHUNK)]],
                               buf1, sem1)
        cp2 = pltpu.async_copy(dst_w.at[idx2_v.at[pl.ds(off, CHUNK)]],
                               buf2, sem2)
        cp1.wait()
        cp2.wait()

        def row_body(r, c):
            for j in range(D_MODEL // LANES):
                sl = pl.ds(j * LANES, LANES)
                buf1[r, sl] = buf1[r, sl] + buf2[r, sl]
            return c

        lax.fori_loop(0, CHUNK, row_body, 0)
        pltpu.sync_copy(buf1, out.at[pl.ds(base + off, CHUNK)])
        return carry

    lax.fori_loop(0, NUM_CHUNKS, chunk_body, 0)


@jax.jit
def _run(src_w, dst_w, mov1, mov2):
    kern = pl.kernel(
        _sc_kernel,
        mesh=plsc.VectorSubcoreMesh(core_axis_name="c", subcore_axis_name="s"),
        out_type=jax.ShapeDtypeStruct((BATCH, D_MODEL), jnp.float32),
        scratch_types=[
            pltpu.VMEM((B_PER_W,), jnp.int32),
            pltpu.VMEM((B_PER_W,), jnp.int32),
            pltpu.VMEM((CHUNK, D_MODEL), jnp.float32),
            pltpu.VMEM((CHUNK, D_MODEL), jnp.float32),
            pltpu.SemaphoreType.DMA,
            pltpu.SemaphoreType.DMA,
        ],
    )
    return kern(src_w, dst_w, mov1, mov2)


def kernel(pieces, mov1, mov2, mov_src_w, mov_dst_w):
    del pieces
    return _run(mov_src_w, mov_dst_w, mov1, mov2)


# speedup vs baseline: 1.2809x; 1.0223x over previous
import jax
import jax.numpy as jnp
from jax import lax
from jax.experimental import pallas as pl
from jax.experimental.pallas import tpu as pltpu
from jax.experimental.pallas import tpu_sc as plsc

D_MODEL = 1024
BATCH = 16384
LANES = 16
NUM_CORES = 2
NUM_SUBCORES = 16
NUM_WORKERS = NUM_CORES * NUM_SUBCORES
B_PER_W = BATCH // NUM_WORKERS
CHUNK = 16
NUM_CHUNKS = B_PER_W // CHUNK
NPAIR = NUM_CHUNKS // 2


def _sc_kernel(src_w, dst_w, mov1, mov2, out, idx1_v, idx2_v,
               buf1a, buf2a, buf1b, buf2b,
               s1a, s2a, s1b, s2b, so):
    wid = lax.axis_index("s") * NUM_CORES + lax.axis_index("c")
    base = wid * B_PER_W

    pltpu.sync_copy(mov1.at[pl.ds(base, B_PER_W)], idx1_v)
    pltpu.sync_copy(mov2.at[pl.ds(base, B_PER_W)], idx2_v)

    def start(k, b1, b2, sg1, sg2):
        off = pl.multiple_of(k * CHUNK, CHUNK)
        pltpu.async_copy(src_w.at[idx1_v.at[pl.ds(off, CHUNK)]], b1, sg1)
        pltpu.async_copy(dst_w.at[idx2_v.at[pl.ds(off, CHUNK)]], b2, sg2)

    def finish(k, b1, b2, sg1, sg2):
        off = pl.multiple_of(k * CHUNK, CHUNK)
        pltpu.make_async_copy(src_w.at[idx1_v.at[pl.ds(off, CHUNK)]], b1,
                              sg1).wait()
        pltpu.make_async_copy(dst_w.at[idx2_v.at[pl.ds(off, CHUNK)]], b2,
                              sg2).wait()

        def row_body(r, c):
            for j in range(D_MODEL // LANES):
                sl = pl.ds(j * LANES, LANES)
                plsc.addupdate(b1.at[r, sl], b2[r, sl])
            return c

        lax.fori_loop(0, CHUNK, row_body, 0)
        pltpu.sync_copy(b1, out.at[pl.ds(base + off, CHUNK)])

    start(0, buf1a, buf2a, s1a, s2a)

    def pair_body(p, carry):
        ka = 2 * p
        start(ka + 1, buf1b, buf2b, s1b, s2b)
        finish(ka, buf1a, buf2a, s1a, s2a)

        def with_next(_):
            start(ka + 2, buf1a, buf2a, s1a, s2a)
            return 0

        lax.cond(p + 1 < NPAIR, with_next, lambda _: 0, 0)
        finish(ka + 1, buf1b, buf2b, s1b, s2b)
        return carry

    lax.fori_loop(0, NPAIR, pair_body, 0)


@jax.jit
def _run(src_w, dst_w, mov1, mov2):
    kern = pl.kernel(
        _sc_kernel,
        mesh=plsc.VectorSubcoreMesh(core_axis_name="c", subcore_axis_name="s"),
        out_type=jax.ShapeDtypeStruct((BATCH, D_MODEL), jnp.float32),
        scratch_types=[
            pltpu.VMEM((B_PER_W,), jnp.int32),
            pltpu.VMEM((B_PER_W,), jnp.int32),
            pltpu.VMEM((CHUNK, D_MODEL), jnp.float32),
            pltpu.VMEM((CHUNK, D_MODEL), jnp.float32),
            pltpu.VMEM((CHUNK, D_MODEL), jnp.float32),
            pltpu.VMEM((CHUNK, D_MODEL), jnp.float32),
            pltpu.SemaphoreType.DMA,
            pltpu.SemaphoreType.DMA,
            pltpu.SemaphoreType.DMA,
            pltpu.SemaphoreType.DMA,
            pltpu.SemaphoreType.DMA,
        ],
    )
    return kern(src_w, dst_w, mov1, mov2)


def kernel(pieces, mov1, mov2, mov_src_w, mov_dst_w):
    del pieces
    return _run(mov_src_w, mov_dst_w, mov1, mov2)


# speedup vs baseline: 2.2666x; 1.7695x over previous
import jax
import jax.numpy as jnp
from jax import lax
from jax.experimental import pallas as pl
from jax.experimental.pallas import tpu as pltpu
from jax.experimental.pallas import tpu_sc as plsc

D_MODEL = 1024
BATCH = 16384
LANES = 16
NUM_CORES = 2
NUM_SUBCORES = 16
NUM_WORKERS = NUM_CORES * NUM_SUBCORES
NUM_BG = 16
BG = BATCH // NUM_BG
DH = D_MODEL // 2
CHUNK = 32
NUM_CHUNKS = BG // CHUNK
NPAIR = NUM_CHUNKS // 2
NVREG = DH // LANES


def _sc_kernel(src_w, dst_w, mov1, mov2, out, t1, t2, idx1_v, idx2_v,
               obuf_a, obuf_b, sa, sb):
    wid = lax.axis_index("s") * NUM_CORES + lax.axis_index("c")
    bg = wid // 2
    h = wid % 2
    row_base = bg * BG
    col = h * DH

    pltpu.sync_copy(src_w.at[:, pl.ds(col, DH)], t1)
    pltpu.sync_copy(dst_w.at[:, pl.ds(col, DH)], t2)
    pltpu.sync_copy(mov1.at[pl.ds(row_base, BG)], idx1_v.at[pl.ds(0, BG)])
    pltpu.sync_copy(mov2.at[pl.ds(row_base, BG)], idx2_v.at[pl.ds(0, BG)])

    def compute_chunk(k, obuf):
        @plsc.parallel_loop(0, CHUNK, unroll=2)
        def row_body(i):
            r1 = idx1_v[pl.ds(k * CHUNK + i, LANES)][0]
            r2 = idx2_v[pl.ds(k * CHUNK + i, LANES)][0]
            for j in range(NVREG):
                sl = pl.ds(j * LANES, LANES)
                obuf[i, sl] = t1[r1, sl] + t2[r2, sl]

    def out_slice(k):
        return out.at[pl.ds(row_base + k * CHUNK, CHUNK), pl.ds(col, DH)]

    def pair_body(p, carry):
        ka = 2 * p

        def wait_prev(_):
            pltpu.make_async_copy(obuf_a, out_slice(ka - 2), sa).wait()
            pltpu.make_async_copy(obuf_b, out_slice(ka - 1), sb).wait()
            return 0

        lax.cond(p > 0, wait_prev, lambda _: 0, 0)
        compute_chunk(ka, obuf_a)
        pltpu.async_copy(obuf_a, out_slice(ka), sa)
        compute_chunk(ka + 1, obuf_b)
        pltpu.async_copy(obuf_b, out_slice(ka + 1), sb)
        return carry

    lax.fori_loop(0, NPAIR, pair_body, 0)
    pltpu.make_async_copy(obuf_a, out_slice(NUM_CHUNKS - 2), sa).wait()
    pltpu.make_async_copy(obuf_b, out_slice(NUM_CHUNKS - 1), sb).wait()


@jax.jit
def _run(src_w, dst_w, mov1, mov2):
    kern = pl.kernel(
        _sc_kernel,
        mesh=plsc.VectorSubcoreMesh(core_axis_name="c", subcore_axis_name="s"),
        out_type=jax.ShapeDtypeStruct((BATCH, D_MODEL), jnp.float32),
        scratch_types=[
            pltpu.VMEM((64, DH), jnp.float32),
            pltpu.VMEM((64, DH), jnp.float32),
            pltpu.VMEM((BG + LANES,), jnp.int32),
            pltpu.VMEM((BG + LANES,), jnp.int32),
            pltpu.VMEM((CHUNK, DH), jnp.float32),
            pltpu.VMEM((CHUNK, DH), jnp.float32),
            pltpu.SemaphoreType.DMA,
            pltpu.SemaphoreType.DMA,
        ],
    )
    return kern(src_w, dst_w, mov1, mov2)


def kernel(pieces, mov1, mov2, mov_src_w, mov_dst_w):
    del pieces
    return _run(mov_src_w, mov_dst_w, mov1, mov2)


# speedup vs baseline: 2.4916x; 1.0993x over previous
import jax
import jax.numpy as jnp
from jax import lax
from jax.experimental import pallas as pl
from jax.experimental.pallas import tpu as pltpu
from jax.experimental.pallas import tpu_sc as plsc

D_MODEL = 1024
BATCH = 16384
LANES = 16
NUM_CORES = 2
NUM_SUBCORES = 16
BG = BATCH // NUM_SUBCORES
DH = D_MODEL // NUM_CORES
CHUNK = 32
NRING = 4
LOOKAHEAD = 2
NUM_CHUNKS = BG // CHUNK
NGROUP = NUM_CHUNKS // NRING
NVREG = DH // LANES
REP_ROWS = NUM_CORES * NUM_SUBCORES * 64


def _sc_kernel(src_w, dst_w, mov1, mov2, out, rep, t1, idx1_v, idx2_v,
               b0, b1, b2, b3, g0, g1, g2, g3, o0, o1, o2, o3):
    bufs = (b0, b1, b2, b3)
    gsems = (g0, g1, g2, g3)
    osems = (o0, o1, o2, o3)
    s = lax.axis_index("s")
    c = lax.axis_index("c")
    row_base = s * BG
    col = c * DH

    pltpu.sync_copy(src_w.at[:, pl.ds(col, DH)], t1)
    pltpu.sync_copy(mov1.at[pl.ds(row_base, BG)], idx1_v.at[pl.ds(0, BG)])
    pltpu.sync_copy(mov2.at[pl.ds(row_base, BG)], idx2_v)

    rep_base = (c * NUM_SUBCORES + s) * 64
    for half in range(2):
        pltpu.sync_copy(dst_w.at[pl.ds(half * 32, 32), pl.ds(col, DH)], b0)
        pltpu.sync_copy(b0, rep.at[pl.ds(rep_base + half * 32, 32)])

    lane_block = c * (NUM_SUBCORES * 64) + lax.iota(jnp.int32, LANES) * 64

    def transform(v, carry):
        sl = pl.ds(v * LANES, LANES)
        idx2_v[sl] = idx2_v[sl] + lane_block
        return carry

    lax.fori_loop(0, BG // LANES, transform, 0)
    plsc.subcore_barrier()

    def gather(k, buf, sem):
        off = pl.multiple_of(k * CHUNK, CHUNK)
        pltpu.async_copy(rep.at[idx2_v.at[pl.ds(off, CHUNK)]], buf, sem)

    def wait_gather(k, buf, sem):
        off = pl.multiple_of(k * CHUNK, CHUNK)
        pltpu.make_async_copy(rep.at[idx2_v.at[pl.ds(off, CHUNK)]], buf,
                              sem).wait()

    def out_slice(k):
        return out.at[pl.ds(row_base + k * CHUNK, CHUNK), pl.ds(col, DH)]

    def add_rows(k, buf):
        @plsc.parallel_loop(0, CHUNK, unroll=2)
        def row_body(i):
            r1 = idx1_v[pl.ds(k * CHUNK + i, LANES)][0]
            for j in range(NVREG):
                sl = pl.ds(j * LANES, LANES)
                plsc.addupdate(buf.at[i, sl], t1[r1, sl])

    gather(0, bufs[0], gsems[0])
    gather(1, bufs[1], gsems[1])

    def group_body(g, carry):
        for p in range(NRING):
            k = g * NRING + p
            kg = k + LOOKAHEAD
            q = (p + LOOKAHEAD) % NRING

            def prep(_):
                def drain(_):
                    pltpu.make_async_copy(bufs[q], out_slice(kg - NRING),
                                          osems[q]).wait()
                    return 0

                lax.cond(kg - NRING >= 0, drain, lambda _: 0, 0)
                gather(kg, bufs[q], gsems[q])
                return 0

            lax.cond(kg < NUM_CHUNKS, prep, lambda _: 0, 0)
            wait_gather(k, bufs[p], gsems[p])
            add_rows(k, bufs[p])
            pltpu.async_copy(bufs[p], out_slice(k), osems[p])
        return carry

    lax.fori_loop(0, NGROUP, group_body, 0)
    for p in range(NRING):
        k = NUM_CHUNKS - NRING + p
        pltpu.make_async_copy(bufs[p], out_slice(k), osems[p]).wait()


@jax.jit
def _run(src_w, dst_w, mov1, mov2):
    kern = pl.kernel(
        _sc_kernel,
        mesh=plsc.VectorSubcoreMesh(core_axis_name="c", subcore_axis_name="s"),
        out_type=jax.ShapeDtypeStruct((BATCH, D_MODEL), jnp.float32),
        scratch_types=[
            pltpu.HBM((REP_ROWS, DH), jnp.float32),
            pltpu.VMEM((64, DH), jnp.float32),
            pltpu.VMEM((BG + LANES,), jnp.int32),
            pltpu.VMEM((BG,), jnp.int32),
            pltpu.VMEM((CHUNK, DH), jnp.float32),
            pltpu.VMEM((CHUNK, DH), jnp.float32),
            pltpu.VMEM((CHUNK, DH), jnp.float32),
            pltpu.VMEM((CHUNK, DH), jnp.float32),
            pltpu.SemaphoreType.DMA,
            pltpu.SemaphoreType.DMA,
            pltpu.SemaphoreType.DMA,
            pltpu.SemaphoreType.DMA,
            pltpu.SemaphoreType.DMA,
            pltpu.SemaphoreType.DMA,
            pltpu.SemaphoreType.DMA,
            pltpu.SemaphoreType.DMA,
        ],
    )
    return kern(src_w, dst_w, mov1, mov2)


def kernel(pieces, mov1, mov2, mov_src_w, mov_dst_w):
    del pieces
    return _run(mov_src_w, mov_dst_w, mov1, mov2)
